# baseline (speedup 1.0000x reference)
import jax
import jax.numpy as jnp
from jax.experimental import pallas as pl

_WIN = 64
_R = 8
_KEEP = _WIN - _R
_WPB = 8


def _window_kernel(x_ref, s_ref, p_ref, wt_ref, xo_ref, so_ref, po_ref):
    xw = x_ref[0]
    srcw = s_ref[0]
    posr = p_ref[0, 0]

    g = jnp.dot(xw, wt_ref[...], preferred_element_type=jnp.float32)
    ncol = jnp.sqrt(jnp.sum(g * g, axis=1, keepdims=True))
    gn = g / jnp.maximum(ncol, 1e-12)
    gnext = jnp.concatenate([gn[1:], gn[-1:]], axis=0)
    simcol = jnp.sum(gn * gnext, axis=1, keepdims=True)

    bits = jax.lax.bitcast_convert_type(srcw, jnp.uint32)
    src_hi = jax.lax.bitcast_convert_type(
        jnp.bitwise_and(bits, jnp.uint32(0xFFFF0000)), jnp.float32)
    src_lo = srcw - src_hi

    jiota = jax.lax.broadcasted_iota(jnp.int32, (1, _WIN), 1)

    simr = simcol.reshape(_WPB, _WIN)
    liota = jax.lax.broadcasted_iota(jnp.int32, (_WPB, _WIN), 1)
    neginf = jnp.float32(-jnp.inf)
    curr = jnp.where(liota < _WIN - 1, simr, neginf)
    psr = jnp.zeros((_WPB, _WIN), dtype=jnp.bool_)
    for _ in range(_R):
        mr = jnp.max(curr, axis=1, keepdims=True)
        idxr = jnp.min(jnp.where(curr == mr, liota, _WIN), axis=1, keepdims=True)
        psr = jnp.logical_or(psr, liota == idxr)
        curr = jnp.where(jnp.abs(liota - idxr) <= 1, neginf, curr)
    secr = jnp.concatenate(
        [jnp.zeros((_WPB, 1), jnp.int32), psr.astype(jnp.int32)[:, :-1]], axis=1)
    cr = secr
    for sft in (1, 2, 4, 8, 16, 32):
        cr = cr + jnp.concatenate(
            [jnp.zeros((_WPB, sft), jnp.int32), cr[:, :-sft]], axis=1)
    keepr = secr == 0
    acc_p = jnp.zeros((_WPB, _KEEP), jnp.int32)
    for d in range(_R + 1):
        m_d = jnp.logical_and(keepr, cr == d)[:, d:d + _KEEP]
        acc_p = acc_p + jnp.where(m_d, posr[:, d:d + _KEEP], 0)
    po_ref[0, 0] = acc_p

    outj_t = jnp.transpose(liota.astype(jnp.float32) - cr.astype(jnp.float32))
    ps_t = jnp.transpose(psr.astype(jnp.float32))
    sec_t = jnp.transpose(secr.astype(jnp.float32))
    tdims = (((0,), (0,)), ((), ()))
    hiprec = jax.lax.Precision.HIGHEST

    jiota_f = jiota.astype(jnp.float32)
    for w in range(_WPB):
        lo = w * _WIN
        n_w = ncol[lo:lo + _WIN]

        qt = (outj_t[:, w:w + 1] == jiota_f).astype(jnp.float32)
        nnext = jnp.concatenate([n_w[1:], n_w[-1:]], axis=0)
        tot = n_w + nnext + 1e-8
        totprev = jnp.concatenate([tot[:1], tot[:-1]], axis=0)
        wv = jnp.where(sec_t[:, w:w + 1] != 0, n_w / totprev,
                       jnp.where(ps_t[:, w:w + 1] != 0, n_w / tot, 1.0))
        wxt = qt * wv

        so_full = (
            jax.lax.dot_general(qt, src_hi[lo:lo + _WIN], tdims,
                                preferred_element_type=jnp.float32)
            + jax.lax.dot_general(qt, src_lo[lo:lo + _WIN], tdims,
                                  preferred_element_type=jnp.float32))
        xo_full = jax.lax.dot_general(wxt, xw[lo:lo + _WIN], tdims,
                                      precision=hiprec,
                                      preferred_element_type=jnp.float32)
        xo_ref[0, w * _KEEP:(w + 1) * _KEEP] = xo_full[:_KEEP]
        so_ref[0, w * _KEEP:(w + 1) * _KEEP] = so_full[:_KEEP]


def kernel(x, source, position_ids, r, window_size, W_group):
    bsz, seq, dim = x.shape
    n_src = source.shape[2]
    nw = seq // _WIN
    ng = nw // _WPB
    rows = _WPB * _WIN
    orows = _WPB * _KEEP
    wt = W_group.T
    pos4 = position_ids.reshape(bsz, ng, _WPB, _WIN)

    xo, so, po = pl.pallas_call(
        _window_kernel,
        grid=(bsz, ng),
        in_specs=[
            pl.BlockSpec((1, rows, dim), lambda b, w: (b, w, 0)),
            pl.BlockSpec((1, rows, n_src), lambda b, w: (b, w, 0)),
            pl.BlockSpec((1, 1, _WPB, _WIN), lambda b, w: (b, w, 0, 0)),
            pl.BlockSpec((dim, _WIN), lambda b, w: (0, 0)),
        ],
        out_specs=[
            pl.BlockSpec((1, orows, dim), lambda b, w: (b, w, 0)),
            pl.BlockSpec((1, orows, n_src), lambda b, w: (b, w, 0)),
            pl.BlockSpec((1, 1, _WPB, _KEEP), lambda b, w: (b, w, 0, 0)),
        ],
        out_shape=[
            jax.ShapeDtypeStruct((bsz, nw * _KEEP, dim), jnp.float32),
            jax.ShapeDtypeStruct((bsz, nw * _KEEP, n_src), jnp.float32),
            jax.ShapeDtypeStruct((bsz, ng, _WPB, _KEEP), jnp.int32),
        ],
    )(x, source, pos4, wt)
    return xo, so, po.reshape(bsz, nw * _KEEP)


# speedup vs baseline: 1.0880x; 1.0880x over previous
import jax
import jax.numpy as jnp
from jax.experimental import pallas as pl

_WIN = 64
_R = 8
_KEEP = _WIN - _R
_WPB = 8


def _plan_window(simcol, kiota):
    neginf = jnp.float32(-jnp.inf)
    cur = jnp.where(kiota < _WIN - 1, simcol, neginf)
    ps = jnp.zeros((_WIN, 1), dtype=jnp.bool_)
    for _ in range(_R):
        m = jnp.max(cur, axis=0, keepdims=True)
        idx = jnp.min(jnp.where(cur == m, kiota, _WIN), axis=0, keepdims=True)
        ps = jnp.logical_or(ps, kiota == idx)
        cur = jnp.where(jnp.abs(kiota - idx) <= 1, neginf, cur)

    psi = ps.astype(jnp.int32)
    second = jnp.concatenate([jnp.zeros((1, 1), jnp.int32), psi[:-1]], axis=0)
    c = second
    for sft in (1, 2, 4, 8, 16, 32):
        c = c + jnp.concatenate(
            [jnp.zeros((sft, 1), jnp.int32), c[:-sft]], axis=0)
    keep = second == 0
    outj = kiota - c
    return ps, second, c, keep, outj


def _window_kernel(x_ref, s_ref, p_ref, wt_ref, xo_ref, so_ref, po_ref):
    xw = x_ref[0]
    srcw = s_ref[0]
    posr = p_ref[0, 0]

    g = jnp.dot(xw, wt_ref[...], preferred_element_type=jnp.float32)
    ncol = jnp.sqrt(jnp.sum(g * g, axis=1, keepdims=True))
    gn = g / jnp.maximum(ncol, 1e-12)
    gnext = jnp.concatenate([gn[1:], gn[-1:]], axis=0)
    simcol = jnp.sum(gn * gnext, axis=1, keepdims=True)

    bits = jax.lax.bitcast_convert_type(srcw, jnp.uint32)
    src_hi = jax.lax.bitcast_convert_type(
        jnp.bitwise_and(bits, jnp.uint32(0xFFFF0000)), jnp.float32)
    src_lo = srcw - src_hi

    kiota = jax.lax.broadcasted_iota(jnp.int32, (_WIN, 1), 0)
    jiota = jax.lax.broadcasted_iota(jnp.int32, (1, _WIN), 1)

    simr = simcol.reshape(_WPB, _WIN)
    liota = jax.lax.broadcasted_iota(jnp.int32, (_WPB, _WIN), 1)
    neginf = jnp.float32(-jnp.inf)
    curr = jnp.where(liota < _WIN - 1, simr, neginf)
    psr = jnp.zeros((_WPB, _WIN), dtype=jnp.bool_)
    for _ in range(_R):
        mr = jnp.max(curr, axis=1, keepdims=True)
        idxr = jnp.min(jnp.where(curr == mr, liota, _WIN), axis=1, keepdims=True)
        psr = jnp.logical_or(psr, liota == idxr)
        curr = jnp.where(jnp.abs(liota - idxr) <= 1, neginf, curr)
    secr = jnp.concatenate(
        [jnp.zeros((_WPB, 1), jnp.int32), psr.astype(jnp.int32)[:, :-1]], axis=1)
    cr = secr
    for sft in (1, 2, 4, 8, 16, 32):
        cr = cr + jnp.concatenate(
            [jnp.zeros((_WPB, sft), jnp.int32), cr[:, :-sft]], axis=1)
    keepr = secr == 0
    acc_p = jnp.zeros((_WPB, _KEEP), jnp.int32)
    for d in range(_R + 1):
        m_d = jnp.logical_and(keepr, cr == d)[:, d:d + _KEEP]
        acc_p = acc_p + jnp.where(m_d, posr[:, d:d + _KEEP], 0)
    po_ref[0, 0] = acc_p
    tdims = (((0,), (0,)), ((), ()))
    hiprec = jax.lax.Precision.HIGHEST

    for w in range(_WPB):
        lo = w * _WIN
        ps, second, c, keep, outj = _plan_window(simcol[lo:lo + _WIN], kiota)
        n_w = ncol[lo:lo + _WIN]

        qt = (outj == jiota).astype(jnp.float32)
        nnext = jnp.concatenate([n_w[1:], n_w[-1:]], axis=0)
        tot = n_w + nnext + 1e-8
        totprev = jnp.concatenate([tot[:1], tot[:-1]], axis=0)
        wv = jnp.where(second != 0, n_w / totprev,
                       jnp.where(ps, n_w / tot, 1.0))
        wxt = qt * wv

        so_full = (
            jax.lax.dot_general(qt, src_hi[lo:lo + _WIN], tdims,
                                preferred_element_type=jnp.float32)
            + jax.lax.dot_general(qt, src_lo[lo:lo + _WIN], tdims,
                                  preferred_element_type=jnp.float32))
        xo_full = jax.lax.dot_general(wxt, xw[lo:lo + _WIN], tdims,
                                      precision=hiprec,
                                      preferred_element_type=jnp.float32)
        xo_ref[0, w * _KEEP:(w + 1) * _KEEP] = xo_full[:_KEEP]
        so_ref[0, w * _KEEP:(w + 1) * _KEEP] = so_full[:_KEEP]


def kernel(x, source, position_ids, r, window_size, W_group):
    bsz, seq, dim = x.shape
    n_src = source.shape[2]
    nw = seq // _WIN
    ng = nw // _WPB
    rows = _WPB * _WIN
    orows = _WPB * _KEEP
    wt = W_group.T
    pos4 = position_ids.reshape(bsz, ng, _WPB, _WIN)

    xo, so, po = pl.pallas_call(
        _window_kernel,
        grid=(bsz, ng),
        in_specs=[
            pl.BlockSpec((1, rows, dim), lambda b, w: (b, w, 0)),
            pl.BlockSpec((1, rows, n_src), lambda b, w: (b, w, 0)),
            pl.BlockSpec((1, 1, _WPB, _WIN), lambda b, w: (b, w, 0, 0)),
            pl.BlockSpec((dim, _WIN), lambda b, w: (0, 0)),
        ],
        out_specs=[
            pl.BlockSpec((1, orows, dim), lambda b, w: (b, w, 0)),
            pl.BlockSpec((1, orows, n_src), lambda b, w: (b, w, 0)),
            pl.BlockSpec((1, 1, _WPB, _KEEP), lambda b, w: (b, w, 0, 0)),
        ],
        out_shape=[
            jax.ShapeDtypeStruct((bsz, nw * _KEEP, dim), jnp.float32),
            jax.ShapeDtypeStruct((bsz, nw * _KEEP, n_src), jnp.float32),
            jax.ShapeDtypeStruct((bsz, ng, _WPB, _KEEP), jnp.int32),
        ],
    )(x, source, pos4, wt)
    return xo, so, po.reshape(bsz, nw * _KEEP)


# speedup vs baseline: 1.1307x; 1.0392x over previous
import jax
import jax.numpy as jnp
from jax.experimental import pallas as pl

_WIN = 64
_R = 8
_KEEP = _WIN - _R
_WPB = 8


def _plan_window(simcol, kiota):
    neginf = jnp.float32(-jnp.inf)
    cur = jnp.where(kiota < _WIN - 1, simcol, neginf)
    ps = jnp.zeros((_WIN, 1), dtype=jnp.bool_)
    for _ in range(_R):
        m = jnp.max(cur, axis=0, keepdims=True)
        idx = jnp.min(jnp.where(cur == m, kiota, _WIN), axis=0, keepdims=True)
        ps = jnp.logical_or(ps, kiota == idx)
        cur = jnp.where(jnp.abs(kiota - idx) <= 1, neginf, cur)

    psi = ps.astype(jnp.int32)
    second = jnp.concatenate([jnp.zeros((1, 1), jnp.int32), psi[:-1]], axis=0)
    c = second
    for sft in (1, 2, 4, 8, 16, 32):
        c = c + jnp.concatenate(
            [jnp.zeros((sft, 1), jnp.int32), c[:-sft]], axis=0)
    keep = second == 0
    outj = kiota - c
    return ps, second, c, keep, outj


def _window_kernel(x_ref, s_ref, p_ref, wt_ref, xo_ref, so_ref, po_ref):
    xw = x_ref[0]
    srcw = s_ref[0]
    posr = p_ref[0, 0]

    g = jnp.dot(xw, wt_ref[...], preferred_element_type=jnp.float32)
    ncol = jnp.sqrt(jnp.sum(g * g, axis=1, keepdims=True))
    gn = g / jnp.maximum(ncol, 1e-12)
    gnext = jnp.concatenate([gn[1:], gn[-1:]], axis=0)
    simcol = jnp.sum(gn * gnext, axis=1, keepdims=True)


    kiota = jax.lax.broadcasted_iota(jnp.int32, (_WIN, 1), 0)
    jiota = jax.lax.broadcasted_iota(jnp.int32, (1, _WIN), 1)

    simr = simcol.reshape(_WPB, _WIN)
    liota = jax.lax.broadcasted_iota(jnp.int32, (_WPB, _WIN), 1)
    neginf = jnp.float32(-jnp.inf)
    curr = jnp.where(liota < _WIN - 1, simr, neginf)
    psr = jnp.zeros((_WPB, _WIN), dtype=jnp.bool_)
    for _ in range(_R):
        mr = jnp.max(curr, axis=1, keepdims=True)
        idxr = jnp.min(jnp.where(curr == mr, liota, _WIN), axis=1, keepdims=True)
        psr = jnp.logical_or(psr, liota == idxr)
        curr = jnp.where(jnp.abs(liota - idxr) <= 1, neginf, curr)
    secr = jnp.concatenate(
        [jnp.zeros((_WPB, 1), jnp.int32), psr.astype(jnp.int32)[:, :-1]], axis=1)
    cr = secr
    for sft in (1, 2, 4, 8, 16, 32):
        cr = cr + jnp.concatenate(
            [jnp.zeros((_WPB, sft), jnp.int32), cr[:, :-sft]], axis=1)
    keepr = secr == 0
    acc_p = jnp.zeros((_WPB, _KEEP), jnp.int32)
    for d in range(_R + 1):
        m_d = jnp.logical_and(keepr, cr == d)[:, d:d + _KEEP]
        acc_p = acc_p + jnp.where(m_d, posr[:, d:d + _KEEP], 0)
    po_ref[0, 0] = acc_p
    tdims = (((0,), (0,)), ((), ()))
    hiprec = jax.lax.Precision.HIGHEST

    for w in range(_WPB):
        lo = w * _WIN
        ps, second, c, keep, outj = _plan_window(simcol[lo:lo + _WIN], kiota)
        n_w = ncol[lo:lo + _WIN]

        qt = (outj == jiota).astype(jnp.float32)
        nnext = jnp.concatenate([n_w[1:], n_w[-1:]], axis=0)
        tot = n_w + nnext + 1e-8
        totprev = jnp.concatenate([tot[:1], tot[:-1]], axis=0)
        wv = jnp.where(second != 0, n_w / totprev,
                       jnp.where(ps, n_w / tot, 1.0))
        wxt = qt * wv

        so_full = jax.lax.dot_general(qt, srcw[lo:lo + _WIN], tdims,
                                      preferred_element_type=jnp.float32)
        xo_full = jax.lax.dot_general(wxt, xw[lo:lo + _WIN], tdims,
                                      precision=hiprec,
                                      preferred_element_type=jnp.float32)
        xo_ref[0, w * _KEEP:(w + 1) * _KEEP] = xo_full[:_KEEP]
        so_ref[0, w * _KEEP:(w + 1) * _KEEP] = so_full[:_KEEP]


def kernel(x, source, position_ids, r, window_size, W_group):
    bsz, seq, dim = x.shape
    n_src = source.shape[2]
    nw = seq // _WIN
    ng = nw // _WPB
    rows = _WPB * _WIN
    orows = _WPB * _KEEP
    wt = W_group.T
    pos4 = position_ids.reshape(bsz, ng, _WPB, _WIN)

    xo, so, po = pl.pallas_call(
        _window_kernel,
        grid=(bsz, ng),
        in_specs=[
            pl.BlockSpec((1, rows, dim), lambda b, w: (b, w, 0)),
            pl.BlockSpec((1, rows, n_src), lambda b, w: (b, w, 0)),
            pl.BlockSpec((1, 1, _WPB, _WIN), lambda b, w: (b, w, 0, 0)),
            pl.BlockSpec((dim, _WIN), lambda b, w: (0, 0)),
        ],
        out_specs=[
            pl.BlockSpec((1, orows, dim), lambda b, w: (b, w, 0)),
            pl.BlockSpec((1, orows, n_src), lambda b, w: (b, w, 0)),
            pl.BlockSpec((1, 1, _WPB, _KEEP), lambda b, w: (b, w, 0, 0)),
        ],
        out_shape=[
            jax.ShapeDtypeStruct((bsz, nw * _KEEP, dim), jnp.float32),
            jax.ShapeDtypeStruct((bsz, nw * _KEEP, n_src), jnp.float32),
            jax.ShapeDtypeStruct((bsz, ng, _WPB, _KEEP), jnp.int32),
        ],
    )(x, source, pos4, wt)
    return xo, so, po.reshape(bsz, nw * _KEEP)
